# Initial kernel scaffold; baseline (speedup 1.0000x reference)
import jax
import jax.numpy as jnp
from jax.experimental import pallas as pl


def kernel(x, edge_index, W1, att_src1, att_dst1, b1, bn_gamma, bn_beta, W2, att_src2, att_dst2, b2):
    raise NotImplementedError("write your pallas kernel here")


# speedup vs baseline: 36.0998x; 36.0998x over previous
import functools

import jax
import jax.numpy as jnp
from jax import lax
from jax.experimental import pallas as pl
from jax.experimental.pallas import tpu as pltpu
from jax.experimental.pallas import tpu_sc as plsc

_N = 10000
_E = 320000
_NEG = 0.2
_NW = 32
_B = 128
_NB = 81
_EP = _NW * _NB * _B
_RPT = 632
_NPAD = 16 * _RPT


def _tc1_body(x_ref, w1_ref, ms_ref, md_ref, exp_ref, node1_ref, adst1_ref):
    h = jnp.dot(x_ref[...], w1_ref[...], preferred_element_type=jnp.float32)
    asrc = jnp.dot(h, ms_ref[...], preferred_element_type=jnp.float32)
    adst = jnp.dot(h, md_ref[...], preferred_element_type=jnp.float32)
    asrc_rep = jnp.dot(asrc, exp_ref[...], preferred_element_type=jnp.float32)
    adst_rep = jnp.dot(adst, exp_ref[...], preferred_element_type=jnp.float32)
    node1_ref[...] = jnp.concatenate([asrc_rep, h], axis=1)
    adst1_ref[...] = adst_rep


def _tc2_body(p_ref, b1_ref, gam_ref, bet_ref, w2_ref, ms2_ref, md2_ref,
              node2_ref, adst2_ref):
    acc = p_ref[0, 0:_N] + p_ref[1, 0:_N]
    denom = acc[:, 0:64]
    h = acc[:, 64:128] / (denom + 1e-16) + b1_ref[...]
    h = jnp.maximum(h, 0.0)
    mean = jnp.mean(h, axis=0, keepdims=True)
    var = jnp.mean((h - mean) * (h - mean), axis=0, keepdims=True)
    h = (h - mean) * lax.rsqrt(var + 1e-5) * gam_ref[...] + bet_ref[...]
    h2 = jnp.dot(h, w2_ref[...], preferred_element_type=jnp.float32)
    a2s = jnp.dot(h2, ms2_ref[...], preferred_element_type=jnp.float32)
    a2d = jnp.dot(h2, md2_ref[...], preferred_element_type=jnp.float32)
    node2_ref[...] = jnp.concatenate([a2s, h2], axis=1)
    adst2_ref[...] = a2d


def _tc3_body(p_ref, b2_ref, out_ref):
    acc = p_ref[0, 0:_N] + p_ref[1, 0:_N]
    out_ref[...] = acc[:, 16:32] / (acc[:, 0:16] + 1e-16) + b2_ref[...]


def _make_edge_body(nj):
    hw = 16 * nj
    row_w = 2 * hw

    def body(node_hbm, adst_hbm, src_hbm, dst_hbm, zrows_hbm, out_hbm,
             srcidx_v, dstidx_v, rows_v, adst_v, msg_v, accum_sh, sem):
        c = lax.axis_index("c")
        s = lax.axis_index("s")
        wid = c * 16 + s
        pltpu.sync_copy(zrows_hbm, accum_sh.at[pl.ds(s * _RPT, _RPT)])
        plsc.subcore_barrier()

        base = wid * (_NB * _B)

        def batch_body(b, carry):
            off = base + b * _B
            pltpu.sync_copy(src_hbm.at[pl.ds(off, _B)], srcidx_v)
            pltpu.sync_copy(dst_hbm.at[pl.ds(off, _B)], dstidx_v.at[0])
            pltpu.async_copy(node_hbm.at[srcidx_v], rows_v, sem).wait()
            pltpu.async_copy(adst_hbm.at[dstidx_v.at[0]], adst_v, sem).wait()

            def edge_body(e, ecarry):
                for j in range(nj):
                    va = rows_v[e, pl.ds(16 * j, 16)]
                    vb = adst_v[e, pl.ds(16 * j, 16)]
                    al = va + vb
                    al = jnp.where(al >= 0.0, al, al * _NEG)
                    ex = jnp.exp(al)
                    msg_v[e, pl.ds(16 * j, 16)] = ex
                    hv = rows_v[e, pl.ds(hw + 16 * j, 16)]
                    msg_v[e, pl.ds(hw + 16 * j, 16)] = hv * ex
                return ecarry

            lax.fori_loop(0, _B, edge_body, 0)
            pltpu.sync_copy(msg_v, accum_sh.at[dstidx_v.at[0]], add=True)
            return carry

        lax.fori_loop(0, _NB, batch_body, 0)
        plsc.subcore_barrier()
        pltpu.sync_copy(accum_sh.at[pl.ds(s * _RPT, _RPT)],
                        out_hbm.at[c, pl.ds(s * _RPT, _RPT)])

    return body, row_w, hw


def _make_edge_kernel(nj):
    body, row_w, hw = _make_edge_body(nj)
    mesh = plsc.VectorSubcoreMesh(core_axis_name="c", subcore_axis_name="s")
    return pl.kernel(
        body,
        out_type=jax.ShapeDtypeStruct((2, _NPAD, row_w), jnp.float32),
        mesh=mesh,
        compiler_params=pltpu.CompilerParams(use_tc_tiling_on_sc=False),
        scratch_types=[
            pltpu.VMEM((_B,), jnp.int32),
            pltpu.VMEM((1, _B), jnp.int32),
            pltpu.VMEM((_B, row_w), jnp.float32),
            pltpu.VMEM((_B, hw), jnp.float32),
            pltpu.VMEM((_B, row_w), jnp.float32),
            pltpu.VMEM_SHARED((_NPAD, row_w), jnp.float32),
            pltpu.SemaphoreType.DMA,
        ],
    )


@jax.jit
def kernel(x, edge_index, W1, att_src1, att_dst1, b1, bn_gamma, bn_beta,
           W2, att_src2, att_dst2, b2):
    f32 = jnp.float32
    loops = jnp.arange(_N, dtype=jnp.int32)
    npad = _EP - _E - _N
    src = jnp.concatenate([edge_index[0], loops,
                           jnp.zeros((npad,), jnp.int32)])
    dst = jnp.concatenate([edge_index[1], loops,
                           jnp.full((npad,), _N, jnp.int32)])

    eye8 = jnp.eye(8, dtype=f32)
    ms1 = jnp.repeat(eye8, 8, axis=0) * att_src1.reshape(64, 1)
    md1 = jnp.repeat(eye8, 8, axis=0) * att_dst1.reshape(64, 1)
    exp8 = jnp.repeat(eye8, 8, axis=1)
    ones16 = jnp.ones((1, 16), f32)
    ms2 = att_src2.reshape(16, 1) * ones16
    md2 = att_dst2.reshape(16, 1) * ones16

    node1, adst1 = pl.pallas_call(
        _tc1_body,
        out_shape=[jax.ShapeDtypeStruct((_N, 128), f32),
                   jax.ShapeDtypeStruct((_N, 64), f32)],
    )(x, W1, ms1, md1, exp8)
    adst1 = jnp.pad(adst1, ((0, 16), (0, 0)))

    z1 = jnp.zeros((_RPT, 128), f32)
    p1 = _make_edge_kernel(4)(node1, adst1, src, dst, z1)

    node2, adst2 = pl.pallas_call(
        _tc2_body,
        out_shape=[jax.ShapeDtypeStruct((_N, 32), f32),
                   jax.ShapeDtypeStruct((_N, 16), f32)],
    )(p1, b1.reshape(1, 64), bn_gamma.reshape(1, 64), bn_beta.reshape(1, 64),
      W2, ms2, md2)
    adst2 = jnp.pad(adst2, ((0, 16), (0, 0)))

    z2 = jnp.zeros((_RPT, 32), f32)
    p2 = _make_edge_kernel(1)(node2, adst2, src, dst, z2)

    out = pl.pallas_call(
        _tc3_body,
        out_shape=jax.ShapeDtypeStruct((_N, 16), f32),
    )(p2, b2.reshape(1, 16))
    return out


# speedup vs baseline: 54.8556x; 1.5196x over previous
import functools

import jax
import jax.numpy as jnp
from jax import lax
from jax.experimental import pallas as pl
from jax.experimental.pallas import tpu as pltpu
from jax.experimental.pallas import tpu_sc as plsc

_N = 10000
_E = 320000
_NEG = 0.2
_NW = 32
_B = 64
_NB = 164
_EP = _NW * _NB * _B
_RPT = 632
_NPAD = 16 * _RPT


def _tc1_body(x_ref, w1_ref, ms_ref, md_ref, exp_ref, node1_ref, adst1_ref):
    h = jnp.dot(x_ref[...], w1_ref[...], preferred_element_type=jnp.float32)
    asrc = jnp.dot(h, ms_ref[...], preferred_element_type=jnp.float32)
    adst = jnp.dot(h, md_ref[...], preferred_element_type=jnp.float32)
    asrc_rep = jnp.dot(asrc, exp_ref[...], preferred_element_type=jnp.float32)
    adst_rep = jnp.dot(adst, exp_ref[...], preferred_element_type=jnp.float32)
    node1_ref[...] = jnp.concatenate([asrc_rep, h], axis=1)
    adst1_ref[...] = adst_rep


def _tc2_body(p_ref, b1_ref, gam_ref, bet_ref, w2_ref, ms2_ref, md2_ref,
              node2_ref, adst2_ref):
    acc = p_ref[0, 0:_N] + p_ref[1, 0:_N]
    denom = acc[:, 0:64]
    h = acc[:, 64:128] / (denom + 1e-16) + b1_ref[...]
    h = jnp.maximum(h, 0.0)
    mean = jnp.mean(h, axis=0, keepdims=True)
    var = jnp.mean((h - mean) * (h - mean), axis=0, keepdims=True)
    h = (h - mean) * lax.rsqrt(var + 1e-5) * gam_ref[...] + bet_ref[...]
    h2 = jnp.dot(h, w2_ref[...], preferred_element_type=jnp.float32)
    a2s = jnp.dot(h2, ms2_ref[...], preferred_element_type=jnp.float32)
    a2d = jnp.dot(h2, md2_ref[...], preferred_element_type=jnp.float32)
    node2_ref[...] = jnp.concatenate([a2s, h2], axis=1)
    adst2_ref[...] = a2d


def _tc3_body(p_ref, b2_ref, out_ref):
    acc = p_ref[0, 0:_N] + p_ref[1, 0:_N]
    out_ref[...] = acc[:, 16:32] / (acc[:, 0:16] + 1e-16) + b2_ref[...]


def _make_edge_body(nj):
    hw = 16 * nj
    row_w = 2 * hw

    def body(node_hbm, adst_hbm, src_hbm, dst_hbm, zrows_hbm, out_hbm,
             sidx_v, dslab_v, rows_v, adst_v, msg_v, accum_sh, ssem, gsem):
        c = lax.axis_index("c")
        s = lax.axis_index("s")
        wid = c * 16 + s
        pltpu.sync_copy(dst_hbm.at[wid], dslab_v)
        pltpu.sync_copy(src_hbm.at[wid, 0], sidx_v.at[0])
        pltpu.sync_copy(src_hbm.at[wid, 1], sidx_v.at[1])
        pltpu.sync_copy(zrows_hbm, accum_sh.at[pl.ds(s * _RPT, _RPT)])
        plsc.subcore_barrier()

        def fetch_src(b, st):
            pltpu.async_copy(src_hbm.at[wid, b], sidx_v.at[st], ssem.at[st])

        def wait_src(st):
            pltpu.make_async_copy(src_hbm.at[0, 0], sidx_v.at[st],
                                  ssem.at[st]).wait()

        def start_gathers(b, st):
            pltpu.async_copy(node_hbm.at[sidx_v.at[st]], rows_v.at[st],
                             gsem.at[st, 0])
            pltpu.async_copy(adst_hbm.at[dslab_v.at[b]], adst_v.at[st],
                             gsem.at[st, 1])

        def wait_gathers(st):
            pltpu.make_async_copy(node_hbm.at[pl.ds(0, _B)], rows_v.at[st],
                                  gsem.at[st, 0]).wait()
            pltpu.make_async_copy(adst_hbm.at[pl.ds(0, _B)], adst_v.at[st],
                                  gsem.at[st, 1]).wait()

        def consume(b, st):
            def edge_body(e, ecarry):
                for j in range(nj):
                    va = rows_v[st, e, pl.ds(16 * j, 16)]
                    vb = adst_v[st, e, pl.ds(16 * j, 16)]
                    al = va + vb
                    al = jnp.maximum(al, al * _NEG)
                    ex = jnp.exp(al)
                    msg_v[e, pl.ds(16 * j, 16)] = ex
                    hv = rows_v[st, e, pl.ds(hw + 16 * j, 16)]
                    msg_v[e, pl.ds(hw + 16 * j, 16)] = hv * ex
                return ecarry

            lax.fori_loop(0, _B, edge_body, 0)
            pltpu.sync_copy(msg_v, accum_sh.at[dslab_v.at[b]], add=True)

        start_gathers(0, 0)
        start_gathers(1, 1)

        def pair_body(g, carry):
            b0 = 2 * g
            wait_gathers(0)
            fetch_src(b0 + 2, 0)
            consume(b0, 0)
            wait_src(0)
            start_gathers(b0 + 2, 0)
            wait_gathers(1)
            fetch_src(b0 + 3, 1)
            consume(b0 + 1, 1)
            wait_src(1)
            start_gathers(b0 + 3, 1)
            return carry

        lax.fori_loop(0, _NB // 2 - 1, pair_body, 0)
        wait_gathers(0)
        consume(_NB - 2, 0)
        wait_gathers(1)
        consume(_NB - 1, 1)
        plsc.subcore_barrier()
        pltpu.sync_copy(accum_sh.at[pl.ds(s * _RPT, _RPT)],
                        out_hbm.at[c, pl.ds(s * _RPT, _RPT)])

    return body, row_w, hw


def _make_edge_kernel(nj):
    body, row_w, hw = _make_edge_body(nj)
    mesh = plsc.VectorSubcoreMesh(core_axis_name="c", subcore_axis_name="s")
    return pl.kernel(
        body,
        out_type=jax.ShapeDtypeStruct((2, _NPAD, row_w), jnp.float32),
        mesh=mesh,
        compiler_params=pltpu.CompilerParams(use_tc_tiling_on_sc=False),
        scratch_types=[
            pltpu.VMEM((2, _B), jnp.int32),
            pltpu.VMEM((_NB, _B), jnp.int32),
            pltpu.VMEM((2, _B, row_w), jnp.float32),
            pltpu.VMEM((2, _B, hw), jnp.float32),
            pltpu.VMEM((_B, row_w), jnp.float32),
            pltpu.VMEM_SHARED((_NPAD, row_w), jnp.float32),
            pltpu.SemaphoreType.DMA((2,)),
            pltpu.SemaphoreType.DMA((2, 2)),
        ],
    )


@jax.jit
def kernel(x, edge_index, W1, att_src1, att_dst1, b1, bn_gamma, bn_beta,
           W2, att_src2, att_dst2, b2):
    f32 = jnp.float32
    loops = jnp.arange(_N, dtype=jnp.int32)
    npad = _EP - _E - _N
    src = jnp.concatenate([edge_index[0], loops,
                           jnp.zeros((npad,), jnp.int32)])
    dst = jnp.concatenate([edge_index[1], loops,
                           jnp.full((npad,), _N, jnp.int32)])
    src = src.reshape(_NW, _NB, _B)
    dst = dst.reshape(_NW, _NB, _B)

    eye8 = jnp.eye(8, dtype=f32)
    ms1 = jnp.repeat(eye8, 8, axis=0) * att_src1.reshape(64, 1)
    md1 = jnp.repeat(eye8, 8, axis=0) * att_dst1.reshape(64, 1)
    exp8 = jnp.repeat(eye8, 8, axis=1)
    ones16 = jnp.ones((1, 16), f32)
    ms2 = att_src2.reshape(16, 1) * ones16
    md2 = att_dst2.reshape(16, 1) * ones16

    node1, adst1 = pl.pallas_call(
        _tc1_body,
        out_shape=[jax.ShapeDtypeStruct((_N, 128), f32),
                   jax.ShapeDtypeStruct((_N, 64), f32)],
    )(x, W1, ms1, md1, exp8)
    adst1 = jnp.pad(adst1, ((0, 16), (0, 0)))

    z1 = jnp.zeros((_RPT, 128), f32)
    p1 = _make_edge_kernel(4)(node1, adst1, src, dst, z1)

    node2, adst2 = pl.pallas_call(
        _tc2_body,
        out_shape=[jax.ShapeDtypeStruct((_N, 32), f32),
                   jax.ShapeDtypeStruct((_N, 16), f32)],
    )(p1, b1.reshape(1, 64), bn_gamma.reshape(1, 64), bn_beta.reshape(1, 64),
      W2, ms2, md2)
    adst2 = jnp.pad(adst2, ((0, 16), (0, 0)))

    z2 = jnp.zeros((_RPT, 32), f32)
    p2 = _make_edge_kernel(1)(node2, adst2, src, dst, z2)

    out = pl.pallas_call(
        _tc3_body,
        out_shape=jax.ShapeDtypeStruct((_N, 16), f32),
    )(p2, b2.reshape(1, 16))
    return out


# speedup vs baseline: 80.2604x; 1.4631x over previous
import functools

import jax
import jax.numpy as jnp
from jax import lax
from jax.experimental import pallas as pl
from jax.experimental.pallas import tpu as pltpu
from jax.experimental.pallas import tpu_sc as plsc

_N = 10000
_E = 320000
_NEG = 0.2
_NW = 32
_EP = 335872
_RPT = 632
_NPAD = 16 * _RPT


def _tc1_body(x_ref, w1_ref, ms_ref, md_ref, exp_ref, node1_ref, adst1_ref):
    h = jnp.dot(x_ref[...], w1_ref[...], preferred_element_type=jnp.float32)
    asrc = jnp.dot(h, ms_ref[...], preferred_element_type=jnp.float32)
    adst = jnp.dot(h, md_ref[...], preferred_element_type=jnp.float32)
    asrc_rep = jnp.dot(asrc, exp_ref[...], preferred_element_type=jnp.float32)
    adst_rep = jnp.dot(adst, exp_ref[...], preferred_element_type=jnp.float32)
    node1_ref[...] = jnp.concatenate([asrc_rep, h], axis=1)
    adst1_ref[...] = adst_rep


def _tc2_body(p_ref, b1_ref, gam_ref, bet_ref, w2_ref, ms2_ref, md2_ref,
              node2_ref, adst2_ref):
    acc = p_ref[0, 0:_N] + p_ref[1, 0:_N]
    denom = acc[:, 0:64]
    h = acc[:, 64:128] / (denom + 1e-16) + b1_ref[...]
    h = jnp.maximum(h, 0.0)
    mean = jnp.mean(h, axis=0, keepdims=True)
    var = jnp.mean((h - mean) * (h - mean), axis=0, keepdims=True)
    h = (h - mean) * lax.rsqrt(var + 1e-5) * gam_ref[...] + bet_ref[...]
    h2 = jnp.dot(h, w2_ref[...], preferred_element_type=jnp.float32)
    a2s = jnp.dot(h2, ms2_ref[...], preferred_element_type=jnp.float32)
    a2d = jnp.dot(h2, md2_ref[...], preferred_element_type=jnp.float32)
    node2_ref[...] = jnp.concatenate([a2s, h2], axis=1)
    adst2_ref[...] = a2d


def _tc3_body(p_ref, b2_ref, out_ref):
    acc = p_ref[0, 0:_N] + p_ref[1, 0:_N]
    out_ref[...] = acc[:, 16:32] / (acc[:, 0:16] + 1e-16) + b2_ref[...]


def _make_edge_body(nj, bb, nb):
    hw = 16 * nj
    row_w = 2 * hw

    def body(node_hbm, adst_hbm, src_hbm, dst_hbm, zrows_hbm, out_hbm,
             sidx_v, dslab_v, rows_v, adst_v, msg_v, accum_sh, ssem, gsem):
        c = lax.axis_index("c")
        s = lax.axis_index("s")
        wid = c * 16 + s
        pltpu.sync_copy(dst_hbm.at[wid], dslab_v)
        pltpu.sync_copy(src_hbm.at[wid, 0], sidx_v.at[0])
        pltpu.sync_copy(src_hbm.at[wid, 1], sidx_v.at[1])
        pltpu.sync_copy(zrows_hbm, accum_sh.at[pl.ds(s * _RPT, _RPT)])
        plsc.subcore_barrier()

        def fetch_src(b, st):
            pltpu.async_copy(src_hbm.at[wid, b], sidx_v.at[st], ssem.at[st])

        def wait_src(st):
            pltpu.make_async_copy(src_hbm.at[0, 0], sidx_v.at[st],
                                  ssem.at[st]).wait()

        def start_gathers(b, st):
            pltpu.async_copy(node_hbm.at[sidx_v.at[st]], rows_v.at[st],
                             gsem.at[st, 0])
            pltpu.async_copy(adst_hbm.at[dslab_v.at[b]], adst_v.at[st],
                             gsem.at[st, 1])

        def wait_gathers(st):
            pltpu.make_async_copy(node_hbm.at[pl.ds(0, bb)], rows_v.at[st],
                                  gsem.at[st, 0]).wait()
            pltpu.make_async_copy(adst_hbm.at[pl.ds(0, bb)], adst_v.at[st],
                                  gsem.at[st, 1]).wait()

        def consume(b, st):
            @plsc.parallel_loop(0, bb, 1, unroll=2)
            def edge_body(e):
                for j in range(nj):
                    va = rows_v[st, e, pl.ds(16 * j, 16)]
                    vb = adst_v[st, e, pl.ds(16 * j, 16)]
                    al = va + vb
                    al = jnp.maximum(al, al * _NEG)
                    ex = jnp.exp(al)
                    msg_v[e, pl.ds(16 * j, 16)] = ex
                    hv = rows_v[st, e, pl.ds(hw + 16 * j, 16)]
                    msg_v[e, pl.ds(hw + 16 * j, 16)] = hv * ex

            pltpu.sync_copy(msg_v, accum_sh.at[dslab_v.at[b]], add=True)

        start_gathers(0, 0)
        start_gathers(1, 1)

        def pair_body(g, carry):
            b0 = 2 * g
            wait_gathers(0)
            fetch_src(b0 + 2, 0)
            consume(b0, 0)
            wait_src(0)
            start_gathers(b0 + 2, 0)
            wait_gathers(1)
            fetch_src(b0 + 3, 1)
            consume(b0 + 1, 1)
            wait_src(1)
            start_gathers(b0 + 3, 1)
            return carry

        lax.fori_loop(0, nb // 2 - 1, pair_body, 0)
        wait_gathers(0)
        consume(nb - 2, 0)
        wait_gathers(1)
        consume(nb - 1, 1)
        plsc.subcore_barrier()
        pltpu.sync_copy(accum_sh.at[pl.ds(s * _RPT, _RPT)],
                        out_hbm.at[c, pl.ds(s * _RPT, _RPT)])

    return body, row_w, hw


def _make_edge_kernel(nj, bb, nb):
    body, row_w, hw = _make_edge_body(nj, bb, nb)
    mesh = plsc.VectorSubcoreMesh(core_axis_name="c", subcore_axis_name="s")
    return pl.kernel(
        body,
        out_type=jax.ShapeDtypeStruct((2, _NPAD, row_w), jnp.float32),
        mesh=mesh,
        compiler_params=pltpu.CompilerParams(use_tc_tiling_on_sc=False),
        scratch_types=[
            pltpu.VMEM((2, bb), jnp.int32),
            pltpu.VMEM((nb, bb), jnp.int32),
            pltpu.VMEM((2, bb, row_w), jnp.float32),
            pltpu.VMEM((2, bb, hw), jnp.float32),
            pltpu.VMEM((bb, row_w), jnp.float32),
            pltpu.VMEM_SHARED((_NPAD, row_w), jnp.float32),
            pltpu.SemaphoreType.DMA((2,)),
            pltpu.SemaphoreType.DMA((2, 2)),
        ],
    )


@jax.jit
def kernel(x, edge_index, W1, att_src1, att_dst1, b1, bn_gamma, bn_beta,
           W2, att_src2, att_dst2, b2):
    f32 = jnp.float32
    loops = jnp.arange(_N, dtype=jnp.int32)
    npad = _EP - _E - _N
    src = jnp.concatenate([edge_index[0], loops,
                           jnp.zeros((npad,), jnp.int32)])
    dst = jnp.concatenate([edge_index[1], loops,
                           jnp.full((npad,), _N, jnp.int32)])
    src1 = src.reshape(_NW, 164, 64)
    dst1 = dst.reshape(_NW, 164, 64)
    src2 = src.reshape(_NW, 82, 128)
    dst2 = dst.reshape(_NW, 82, 128)

    eye8 = jnp.eye(8, dtype=f32)
    ms1 = jnp.repeat(eye8, 8, axis=0) * att_src1.reshape(64, 1)
    md1 = jnp.repeat(eye8, 8, axis=0) * att_dst1.reshape(64, 1)
    exp8 = jnp.repeat(eye8, 8, axis=1)
    ones16 = jnp.ones((1, 16), f32)
    ms2 = att_src2.reshape(16, 1) * ones16
    md2 = att_dst2.reshape(16, 1) * ones16

    node1, adst1 = pl.pallas_call(
        _tc1_body,
        out_shape=[jax.ShapeDtypeStruct((_N, 128), f32),
                   jax.ShapeDtypeStruct((_N, 64), f32)],
    )(x, W1, ms1, md1, exp8)
    adst1 = jnp.pad(adst1, ((0, 16), (0, 0)))

    z1 = jnp.zeros((_RPT, 128), f32)
    p1 = _make_edge_kernel(4, 64, 164)(node1, adst1, src1, dst1, z1)

    node2, adst2 = pl.pallas_call(
        _tc2_body,
        out_shape=[jax.ShapeDtypeStruct((_N, 32), f32),
                   jax.ShapeDtypeStruct((_N, 16), f32)],
    )(p1, b1.reshape(1, 64), bn_gamma.reshape(1, 64), bn_beta.reshape(1, 64),
      W2, ms2, md2)
    adst2 = jnp.pad(adst2, ((0, 16), (0, 0)))

    z2 = jnp.zeros((_RPT, 32), f32)
    p2 = _make_edge_kernel(1, 128, 82)(node2, adst2, src2, dst2, z2)

    out = pl.pallas_call(
        _tc3_body,
        out_shape=jax.ShapeDtypeStruct((_N, 16), f32),
    )(p2, b2.reshape(1, 16))
    return out
